# baseline (speedup 1.0000x reference)
import functools

import jax
import jax.numpy as jnp
from jax import lax
from jax.experimental import pallas as pl
from jax.experimental.pallas import tpu as pltpu
from jax.experimental.pallas import tpu_sc as plsc

N = 10000
E = 320000
D = 128
H = 128
G = 20
K = 16
BINS = 16

NP = 10240
NC, NS = 2, 16
CH = 128
EPT = 20096
EPAD = NS * EPT
NCHUNK = EPT // CH
NCH2 = 158
NCHS = 160
RPT = NP // NS

_mesh = functools.partial(
    plsc.VectorSubcoreMesh, core_axis_name="c", subcore_axis_name="s",
    num_cores=NC, num_subcores=NS)


@functools.cache
def _make_sc_deg():
    @functools.partial(
        pl.kernel,
        out_type=jax.ShapeDtypeStruct((NC * NS * NP,), jnp.float32),
        mesh=_mesh(),
        compiler_params=pltpu.CompilerParams(needs_layout_passes=False),
        scratch_types=[
            pltpu.VMEM((NCHS, CH), jnp.int32),
            pltpu.VMEM((NP,), jnp.float32),
        ],
    )
    def _sc_deg(dst3_hbm, deg_hbm, dstb_v, degtab_v):
        c = lax.axis_index("c")
        s = lax.axis_index("s")
        w = c * NS + s

        zero16 = jnp.zeros((16,), jnp.float32)

        def _zero(i, _):
            degtab_v[pl.ds(i * 16, 16)] = zero16
            return 0
        lax.fori_loop(0, NP // 16, _zero, 0)

        pltpu.sync_copy(dst3_hbm.at[w], dstb_v)
        one16 = jnp.full((16,), 1.0, jnp.float32)

        def _chunk(i, _):
            def _inner(j, _):
                d16 = dstb_v[i, pl.ds(j * 16, 16)]
                plsc.addupdate_scatter(degtab_v, [d16], one16)
                return 0
            lax.fori_loop(0, CH // 16, _inner, 0)
            return 0
        lax.fori_loop(0, NCHUNK, _chunk, 0)

        pltpu.sync_copy(degtab_v, deg_hbm.at[pl.ds((c * NS + s) * NP, NP)])

    return _sc_deg


@functools.cache
def _make_sc_agg():
    @functools.partial(
        pl.kernel,
        out_type=jax.ShapeDtypeStruct((NC * NP, H), jnp.float32),
        mesh=_mesh(),
        compiler_params=pltpu.CompilerParams(needs_layout_passes=False),
        scratch_types=[
            pltpu.VMEM((1, CH), jnp.int32),
            pltpu.VMEM((1, CH), jnp.int32),
            pltpu.VMEM((1, CH), jnp.int32),
            pltpu.VMEM((1, CH), jnp.int32),
            pltpu.VMEM((CH, H), jnp.float32),
            pltpu.VMEM((CH, H), jnp.float32),
            pltpu.VMEM_SHARED((NP, H), jnp.float32),
            pltpu.SemaphoreType.DMA,
            pltpu.SemaphoreType.DMA,
            pltpu.SemaphoreType.DMA,
            pltpu.SemaphoreType.DMA,
            pltpu.SemaphoreType.DMA,
            pltpu.SemaphoreType.DMA,
        ],
    )
    def _sc_agg(src3_hbm, dst3_hbm, hp_hbm, out_hbm, srcA, dstA, srcB, dstB,
                rowsA, rowsB, acc_sh, isemA, isemB, gsemA, gsemB,
                ssemA, ssemB):
        c = lax.axis_index("c")
        s = lax.axis_index("s")
        w = c * NS + s

        zero16 = jnp.zeros((16,), jnp.float32)

        def _zrow(i, _):
            for l in range(H // 16):
                rowsA[i, pl.ds(l * 16, 16)] = zero16
            return 0
        lax.fori_loop(0, CH, _zrow, 0)
        for r in range(RPT // CH):
            pltpu.sync_copy(rowsA, acc_sh.at[pl.ds(s * RPT + r * CH, CH)])
        plsc.subcore_barrier()

        def fetch_idx(i, sb, db, sem):
            pltpu.async_copy(src3_hbm.at[w, pl.ds(i, 1)], sb, sem)
            pltpu.async_copy(dst3_hbm.at[w, pl.ds(i, 1)], db, sem)

        def wait_idx(i, sb, db, sem):
            pltpu.make_async_copy(src3_hbm.at[w, pl.ds(i, 1)], sb, sem).wait()
            pltpu.make_async_copy(dst3_hbm.at[w, pl.ds(i, 1)], db, sem).wait()

        wait = pltpu.make_async_copy
        fetch_idx(0, srcA, dstA, isemA)
        fetch_idx(1, srcB, dstB, isemB)
        wait_idx(0, srcA, dstA, isemA)
        pltpu.async_copy(hp_hbm.at[srcA.at[0]], rowsA, gsemA)
        wait_idx(1, srcB, dstB, isemB)
        pltpu.async_copy(hp_hbm.at[srcB.at[0]], rowsB, gsemB)

        def _chunk2(i2, _):
            i = 2 * i2
            wait(hp_hbm.at[srcA.at[0]], rowsA, gsemA).wait()
            pltpu.async_copy(rowsA, acc_sh.at[dstA.at[0]], ssemA, add=True)
            wait(hp_hbm.at[srcB.at[0]], rowsB, gsemB).wait()
            pltpu.async_copy(rowsB, acc_sh.at[dstB.at[0]], ssemB, add=True)
            wait(rowsA, acc_sh.at[dstA.at[0]], ssemA).wait()
            fetch_idx(i + 2, srcA, dstA, isemA)
            wait(rowsB, acc_sh.at[dstB.at[0]], ssemB).wait()
            fetch_idx(i + 3, srcB, dstB, isemB)
            wait_idx(i + 2, srcA, dstA, isemA)
            pltpu.async_copy(hp_hbm.at[srcA.at[0]], rowsA, gsemA)
            wait_idx(i + 3, srcB, dstB, isemB)
            pltpu.async_copy(hp_hbm.at[srcB.at[0]], rowsB, gsemB)
            return 0
        lax.fori_loop(0, NCH2 // 2, _chunk2, 0)

        wait(hp_hbm.at[srcA.at[0]], rowsA, gsemA).wait()
        wait(hp_hbm.at[srcB.at[0]], rowsB, gsemB).wait()

        plsc.subcore_barrier()
        pltpu.sync_copy(acc_sh.at[pl.ds(s * RPT, RPT)],
                        out_hbm.at[pl.ds(c * NP + s * RPT, RPT)])

    return _sc_agg


_RB = 2048
_NRB = NP // _RB


def _tc_prep_body(x_ref, tab_ref, w_ref, hp_ref, dinvb_ref):
    tab = tab_ref[0]
    ones = jnp.ones((NS, 1), jnp.float32)
    deg = 1.0 + lax.dot_general(tab, ones, (((0,), (0,)), ((), ())),
                                preferred_element_type=jnp.float32)
    dinv = lax.rsqrt(deg)
    dinvb = jnp.broadcast_to(dinv, (_RB, H))
    h = lax.dot_general(x_ref[0], w_ref[...], (((1,), (1,)), ((), ())),
                        preferred_element_type=jnp.float32)
    hp_ref[0] = h * dinvb
    dinvb_ref[0] = dinvb


def _tc_prep(xs, degtab, w0):
    return pl.pallas_call(
        _tc_prep_body,
        grid=(NC, _NRB),
        in_specs=[
            pl.BlockSpec((1, _RB, D), lambda c, i: (c, i, 0)),
            pl.BlockSpec((1, NS, _RB), lambda c, i: (c, 0, i)),
            pl.BlockSpec((H, D), lambda c, i: (0, 0)),
        ],
        out_specs=[
            pl.BlockSpec((1, _RB, H), lambda c, i: (c, i, 0)),
            pl.BlockSpec((1, _RB, H), lambda c, i: (c, i, 0)),
        ],
        out_shape=[
            jax.ShapeDtypeStruct((NC, NP, H), jnp.float32),
            jax.ShapeDtypeStruct((NC, NP, H), jnp.float32),
        ],
    )(xs, degtab, w0)


def _bn_relu(h, vec):
    h = h + vec[0:1, :]
    h = (h - vec[1:2, :]) * lax.rsqrt(vec[2:3, :] + 1e-5) * vec[3:4, :] \
        + vec[4:5, :]
    return jnp.maximum(h, 0.0)


def _tc_mid_body(agg_ref, hp_ref, dinvb_ref, vec_ref, w_ref, out_ref):
    dinvb = dinvb_ref[0]
    h = (agg_ref[0] + hp_ref[0]) * dinvb
    h = _bn_relu(h, vec_ref[...])
    hn = lax.dot_general(h, w_ref[...], (((1,), (1,)), ((), ())),
                         preferred_element_type=jnp.float32)
    out_ref[0] = hn * dinvb


def _tc_mid(agg, hp, dinvb, vec, w):
    return pl.pallas_call(
        _tc_mid_body,
        grid=(NC, _NRB),
        in_specs=[
            pl.BlockSpec((1, _RB, H), lambda c, i: (c, i, 0)),
            pl.BlockSpec((1, _RB, H), lambda c, i: (c, i, 0)),
            pl.BlockSpec((1, _RB, H), lambda c, i: (c, i, 0)),
            pl.BlockSpec((8, H), lambda c, i: (0, 0)),
            pl.BlockSpec((H, H), lambda c, i: (0, 0)),
        ],
        out_specs=pl.BlockSpec((1, _RB, H), lambda c, i: (c, i, 0)),
        out_shape=jax.ShapeDtypeStruct((NC, NP, H), jnp.float32),
    )(agg, hp, dinvb, vec, w)


_SEG = N // G


def _tc_final_body(agg_ref, hp_ref, dinvb_ref, vec_ref, wc_ref, avec_ref,
                   g_ref, an_ref):
    e = (agg_ref[0, 0] + hp_ref[0, 0]) * dinvb_ref[0, 0]
    e = _bn_relu(e, vec_ref[...])
    t = lax.dot_general(e, wc_ref[...], (((1,), (1,)), ((), ())),
                        preferred_element_type=jnp.float32)
    t = jnp.tanh(t + avec_ref[0:1, :])
    s = jnp.sum(t * avec_ref[1:2, :], axis=1, keepdims=True) + avec_ref[2, 0]
    m = jnp.max(s)
    w = jnp.exp(s - m)
    w = w / jnp.sum(w)
    gvec = jnp.sum(w * e, axis=0, keepdims=True)
    g_ref[0, 0] = jnp.broadcast_to(gvec, (8, H))
    nrm = jnp.sqrt(jnp.sum(e * e, axis=1, keepdims=True))
    an_ref[0, 0] = e / jnp.maximum(nrm, 1e-8)


def _tc_final(agg, hp, dinvb, vec, wc, avec):
    return pl.pallas_call(
        _tc_final_body,
        grid=(NC, G),
        in_specs=[
            pl.BlockSpec((1, 1, _SEG, H), lambda c, g: (c, g, 0, 0)),
            pl.BlockSpec((1, 1, _SEG, H), lambda c, g: (c, g, 0, 0)),
            pl.BlockSpec((1, 1, _SEG, H), lambda c, g: (c, g, 0, 0)),
            pl.BlockSpec((8, H), lambda c, g: (0, 0)),
            pl.BlockSpec((H, H), lambda c, g: (0, 0)),
            pl.BlockSpec((8, H), lambda c, g: (0, 0)),
        ],
        out_specs=[
            pl.BlockSpec((1, 1, 8, H), lambda c, g: (c, g, 0, 0)),
            pl.BlockSpec((1, 1, _SEG, H), lambda c, g: (c, g, 0, 0)),
        ],
        out_shape=[
            jax.ShapeDtypeStruct((NC, G, 8, H), jnp.float32),
            jax.ShapeDtypeStruct((NC, G, _SEG, H), jnp.float32),
        ],
    )(agg, hp, dinvb, vec, wc, avec)


def _tc_hist_body(a1_ref, a2_ref, hist_ref):
    sims = lax.dot_general(a1_ref[0, 0], a2_ref[0, 0],
                           (((1,), (1,)), ((), ())),
                           preferred_element_type=jnp.float32)
    u = (sims + 1.0) * (BINS / 2.0)
    q = jnp.clip(jnp.floor(u), 0.0, BINS - 1.0)
    lanes = lax.broadcasted_iota(jnp.int32, (1, H), 1)
    acc = jnp.zeros((1, H), jnp.float32)
    total = 0.0
    for b in range(BINS):
        cnt = jnp.sum(jnp.where(q == float(b), 1.0, 0.0))
        acc = acc + cnt * jnp.where(lanes == b, 1.0, 0.0)
        total = total + cnt
    hist_ref[0] = jnp.broadcast_to(acc / (total + 1e-8), (8, H))


def _tc_hist(an):
    return pl.pallas_call(
        _tc_hist_body,
        grid=(G,),
        in_specs=[
            pl.BlockSpec((1, 1, _SEG, H), lambda g: (0, g, 0, 0)),
            pl.BlockSpec((1, 1, _SEG, H), lambda g: (1, g, 0, 0)),
        ],
        out_specs=pl.BlockSpec((1, 8, H), lambda g: (g, 0, 0)),
        out_shape=jax.ShapeDtypeStruct((G, 8, H), jnp.float32),
    )(an, an)


def _tc_tail_body(g1_ref, g2_ref, hist_ref, T_ref, nb_ref, m1w_ref, m1b_ref,
                  m2w_ref, m2b_ref, m3w_ref, m3b_ref, h1w_ref, h1b_ref,
                  h2w_ref, h2b_ref, f1w_ref, f1b_ref, f2w_ref, f2b_ref,
                  out_ref):
    g1 = g1_ref[...]
    g2 = g2_ref[...]
    lanes16 = lax.broadcasted_iota(jnp.int32, (1, K), 1)
    tf = jnp.zeros((G, K), jnp.float32)
    for k in range(K):
        vk = lax.dot_general(g1, T_ref[k], (((1,), (0,)), ((), ())),
                             preferred_element_type=jnp.float32)
        col = jnp.sum(vk * g2, axis=1, keepdims=True)
        tf = tf + col * jnp.where(lanes16 == k, 1.0, 0.0)
    tf = tf + nb_ref[0:1, :]
    h = jnp.maximum(lax.dot_general(tf, m1w_ref[...], (((1,), (1,)), ((), ())),
                                    preferred_element_type=jnp.float32)
                    + m1b_ref[0:1, :], 0.0)
    h = jnp.maximum(lax.dot_general(h, m2w_ref[...], (((1,), (1,)), ((), ())),
                                    preferred_element_type=jnp.float32)
                    + m2b_ref[0:1, :], 0.0)
    ntn = jax.nn.sigmoid(jnp.sum(h * m3w_ref[0:1, :], axis=1, keepdims=True)
                         + m3b_ref[0, 0])
    hh = jnp.maximum(lax.dot_general(hist_ref[...], h1w_ref[...],
                                     (((1,), (1,)), ((), ())),
                                     preferred_element_type=jnp.float32)
                     + h1b_ref[0:1, :], 0.0)
    hemb = lax.dot_general(hh, h2w_ref[...], (((1,), (1,)), ((), ())),
                           preferred_element_type=jnp.float32) + h2b_ref[0:1, :]
    f1wt = f1w_ref[...]
    f = ntn * f1wt[0:1, :] \
        + lax.dot_general(hemb, lax.slice(f1wt, (1, 0), (17, 8)),
                          (((1,), (0,)), ((), ())),
                          preferred_element_type=jnp.float32)
    f = jnp.maximum(f + f1b_ref[0:1, :], 0.0)
    out = jax.nn.sigmoid(jnp.sum(f * f2w_ref[0:1, :], axis=1, keepdims=True)
                         + f2b_ref[0, 0])
    out_ref[...] = out


def _tc_tail(g1, g2, hist, p):
    args = (g1, g2, hist, p['ntn_T'], p['ntn_bias'][None, :],
            p['ntn_m1_W'], p['ntn_m1_b'][None, :],
            p['ntn_m2_W'], p['ntn_m2_b'][None, :],
            p['ntn_m3_W'], p['ntn_m3_b'][None, None],
            p['hist1_W'], p['hist1_b'][None, :],
            p['hist2_W'], p['hist2_b'][None, :],
            p['fus1_W'].T, p['fus1_b'][None, :],
            p['fus2_W'], p['fus2_b'][None, None])
    return pl.pallas_call(
        _tc_tail_body,
        out_shape=jax.ShapeDtypeStruct((G, 1), jnp.float32),
    )(*args)


def _stage_edges(ei, row_off):
    src = jnp.concatenate(
        [ei[0] + row_off,
         jnp.full((EPAD - E,), row_off, jnp.int32)]).reshape(NS, NCHUNK, CH)
    dst = jnp.concatenate(
        [ei[1], jnp.full((EPAD - E,), NP - 8, jnp.int32)]
    ).reshape(NS, NCHUNK, CH)
    pad = NCHS - NCHUNK
    src = jnp.concatenate(
        [src, jnp.full((NS, pad, CH), row_off, jnp.int32)], axis=1)
    dst = jnp.concatenate(
        [dst, jnp.full((NS, pad, CH), NP - 8, jnp.int32)], axis=1)
    return src, dst


def kernel(x1, edge_index1, batch1, x2, edge_index2, batch2, params):
    p = params
    s1, d1 = _stage_edges(edge_index1, 0)
    s2, d2 = _stage_edges(edge_index2, NP)
    src3 = jnp.concatenate([s1, s2])
    dst3 = jnp.concatenate([d1, d2])

    xs = jnp.zeros((NC, NP, D), jnp.float32)
    xs = xs.at[0, :N].set(x1).at[1, :N].set(x2)

    degtab = _make_sc_deg()(dst3).reshape(NC, NS, NP)

    def vecs(l):
        return jnp.stack([
            p['conv%d_b' % l], p['bn%d_mean' % l], p['bn%d_var' % l],
            p['bn%d_gamma' % l], p['bn%d_beta' % l],
            jnp.zeros((H,), jnp.float32), jnp.zeros((H,), jnp.float32),
            jnp.zeros((H,), jnp.float32)])

    hp0, dinvb = _tc_prep(xs, degtab, p['conv0_W'])

    def agg(hp):
        return _make_sc_agg()(src3, dst3,
                              hp.reshape(NC * NP, H)).reshape(NC, NP, H)

    agg0 = agg(hp0)
    hp1 = _tc_mid(agg0, hp0, dinvb, vecs(0), p['conv1_W'])
    agg1 = agg(hp1)
    hp2 = _tc_mid(agg1, hp1, dinvb, vecs(1), p['conv2_W'])
    agg2 = agg(hp2)

    avec = jnp.zeros((8, H), jnp.float32)
    avec = avec.at[0].set(p['attn_cw_b']).at[1].set(p['attn_a_W'][0])
    avec = avec.at[2, 0].set(p['attn_a_b'][0])

    def seg4(a):
        return a[:, :N].reshape(NC, G, _SEG, H)
    gout, an = _tc_final(seg4(agg2), seg4(hp2), seg4(dinvb), vecs(2),
                         p['attn_cw_W'], avec)

    hist = _tc_hist(an)[:, 0, :BINS]
    g1 = gout[0, :, 0, :]
    g2 = gout[1, :, 0, :]
    return _tc_tail(g1, g2, hist, p)


# speedup vs baseline: 1.0683x; 1.0683x over previous
import functools

import jax
import jax.numpy as jnp
from jax import lax
from jax.experimental import pallas as pl
from jax.experimental.pallas import tpu as pltpu
from jax.experimental.pallas import tpu_sc as plsc

N = 10000
E = 320000
D = 128
H = 128
G = 20
K = 16
BINS = 16

NP = 10240
NC, NS = 2, 16
CH = 128
EPT = 20096
EPAD = NS * EPT
NCHUNK = EPT // CH
NCH2 = 158
NCHS = 160
RPT = NP // NS

_mesh = functools.partial(
    plsc.VectorSubcoreMesh, core_axis_name="c", subcore_axis_name="s",
    num_cores=NC, num_subcores=NS)


@functools.cache
def _make_sc_deg():
    @functools.partial(
        pl.kernel,
        out_type=jax.ShapeDtypeStruct((NC * NS * NP,), jnp.float32),
        mesh=_mesh(),
        compiler_params=pltpu.CompilerParams(needs_layout_passes=False),
        scratch_types=[
            pltpu.VMEM((NCHS, CH), jnp.int32),
            pltpu.VMEM((NP,), jnp.float32),
        ],
    )
    def _sc_deg(dst3_hbm, deg_hbm, dstb_v, degtab_v):
        c = lax.axis_index("c")
        s = lax.axis_index("s")
        w = c * NS + s

        zero16 = jnp.zeros((16,), jnp.float32)

        def _zero(i, _):
            degtab_v[pl.ds(i * 16, 16)] = zero16
            return 0
        lax.fori_loop(0, NP // 16, _zero, 0)

        pltpu.sync_copy(dst3_hbm.at[w], dstb_v)
        one16 = jnp.full((16,), 1.0, jnp.float32)

        def _chunk(i, _):
            def _inner(j, _):
                d16 = dstb_v[i, pl.ds(j * 16, 16)]
                plsc.addupdate_scatter(degtab_v, [d16], one16)
                return 0
            lax.fori_loop(0, CH // 16, _inner, 0)
            return 0
        lax.fori_loop(0, NCHUNK, _chunk, 0)

        pltpu.sync_copy(degtab_v, deg_hbm.at[pl.ds((c * NS + s) * NP, NP)])

    return _sc_deg


@functools.cache
def _make_sc_agg():
    @functools.partial(
        pl.kernel,
        out_type=jax.ShapeDtypeStruct((NC * NP, H), jnp.float32),
        mesh=_mesh(),
        compiler_params=pltpu.CompilerParams(needs_layout_passes=False),
        scratch_types=[
            pltpu.VMEM((1, CH), jnp.int32),
            pltpu.VMEM((1, CH), jnp.int32),
            pltpu.VMEM((1, CH), jnp.int32),
            pltpu.VMEM((1, CH), jnp.int32),
            pltpu.VMEM((CH, H), jnp.float32),
            pltpu.VMEM((CH, H), jnp.float32),
            pltpu.VMEM_SHARED((NP, H), jnp.float32),
            pltpu.SemaphoreType.DMA,
            pltpu.SemaphoreType.DMA,
            pltpu.SemaphoreType.DMA,
            pltpu.SemaphoreType.DMA,
        ],
    )
    def _sc_agg(src3_hbm, dst3_hbm, hp_hbm, out_hbm, srcA, dstA, srcB, dstB,
                rowsA, rowsB, acc_sh, isemA, isemB, gsemA, gsemB):
        c = lax.axis_index("c")
        s = lax.axis_index("s")
        w = c * NS + s

        zero16 = jnp.zeros((16,), jnp.float32)

        def _zrow(i, _):
            for l in range(H // 16):
                rowsA[i, pl.ds(l * 16, 16)] = zero16
            return 0
        lax.fori_loop(0, CH, _zrow, 0)
        for r in range(RPT // CH):
            pltpu.sync_copy(rowsA, acc_sh.at[pl.ds(s * RPT + r * CH, CH)])
        plsc.subcore_barrier()

        def fetch_idx(i, sb, db, sem):
            pltpu.async_copy(src3_hbm.at[w, pl.ds(i, 1)], sb, sem)
            pltpu.async_copy(dst3_hbm.at[w, pl.ds(i, 1)], db, sem)

        def wait_idx(i, sb, db, sem):
            pltpu.make_async_copy(src3_hbm.at[w, pl.ds(i, 1)], sb, sem).wait()
            pltpu.make_async_copy(dst3_hbm.at[w, pl.ds(i, 1)], db, sem).wait()

        wait = pltpu.make_async_copy
        fetch_idx(0, srcA, dstA, isemA)
        fetch_idx(1, srcB, dstB, isemB)
        wait_idx(0, srcA, dstA, isemA)
        pltpu.async_copy(hp_hbm.at[srcA.at[0]], rowsA, gsemA)
        wait_idx(1, srcB, dstB, isemB)
        pltpu.async_copy(hp_hbm.at[srcB.at[0]], rowsB, gsemB)

        def _chunk2(i2, _):
            i = 2 * i2
            wait(hp_hbm.at[srcA.at[0]], rowsA, gsemA).wait()
            pltpu.sync_copy(rowsA, acc_sh.at[dstA.at[0]], add=True)
            fetch_idx(i + 2, srcA, dstA, isemA)
            wait(hp_hbm.at[srcB.at[0]], rowsB, gsemB).wait()
            pltpu.sync_copy(rowsB, acc_sh.at[dstB.at[0]], add=True)
            fetch_idx(i + 3, srcB, dstB, isemB)
            wait_idx(i + 2, srcA, dstA, isemA)
            pltpu.async_copy(hp_hbm.at[srcA.at[0]], rowsA, gsemA)
            wait_idx(i + 3, srcB, dstB, isemB)
            pltpu.async_copy(hp_hbm.at[srcB.at[0]], rowsB, gsemB)
            return 0
        lax.fori_loop(0, NCH2 // 2, _chunk2, 0)

        wait(hp_hbm.at[srcA.at[0]], rowsA, gsemA).wait()
        wait(hp_hbm.at[srcB.at[0]], rowsB, gsemB).wait()

        plsc.subcore_barrier()
        pltpu.sync_copy(acc_sh.at[pl.ds(s * RPT, RPT)],
                        out_hbm.at[pl.ds(c * NP + s * RPT, RPT)])

    return _sc_agg


_RB = 2048
_NRB = NP // _RB


def _tc_prep_body(x_ref, tab_ref, w_ref, hp_ref, dinvb_ref):
    tab = tab_ref[0]
    ones = jnp.ones((NS, 1), jnp.float32)
    deg = 1.0 + lax.dot_general(tab, ones, (((0,), (0,)), ((), ())),
                                preferred_element_type=jnp.float32)
    dinv = lax.rsqrt(deg)
    dinvb = jnp.broadcast_to(dinv, (_RB, H))
    h = lax.dot_general(x_ref[0], w_ref[...], (((1,), (1,)), ((), ())),
                        preferred_element_type=jnp.float32)
    hp_ref[0] = h * dinvb
    dinvb_ref[0] = dinvb


def _tc_prep(xs, degtab, w0):
    return pl.pallas_call(
        _tc_prep_body,
        grid=(NC, _NRB),
        in_specs=[
            pl.BlockSpec((1, _RB, D), lambda c, i: (c, i, 0)),
            pl.BlockSpec((1, NS, _RB), lambda c, i: (c, 0, i)),
            pl.BlockSpec((H, D), lambda c, i: (0, 0)),
        ],
        out_specs=[
            pl.BlockSpec((1, _RB, H), lambda c, i: (c, i, 0)),
            pl.BlockSpec((1, _RB, H), lambda c, i: (c, i, 0)),
        ],
        out_shape=[
            jax.ShapeDtypeStruct((NC, NP, H), jnp.float32),
            jax.ShapeDtypeStruct((NC, NP, H), jnp.float32),
        ],
    )(xs, degtab, w0)


def _bn_relu(h, vec):
    h = h + vec[0:1, :]
    h = (h - vec[1:2, :]) * lax.rsqrt(vec[2:3, :] + 1e-5) * vec[3:4, :] \
        + vec[4:5, :]
    return jnp.maximum(h, 0.0)


def _tc_mid_body(agg_ref, hp_ref, dinvb_ref, vec_ref, w_ref, out_ref):
    dinvb = dinvb_ref[0]
    h = (agg_ref[0] + hp_ref[0]) * dinvb
    h = _bn_relu(h, vec_ref[...])
    hn = lax.dot_general(h, w_ref[...], (((1,), (1,)), ((), ())),
                         preferred_element_type=jnp.float32)
    out_ref[0] = hn * dinvb


def _tc_mid(agg, hp, dinvb, vec, w):
    return pl.pallas_call(
        _tc_mid_body,
        grid=(NC, _NRB),
        in_specs=[
            pl.BlockSpec((1, _RB, H), lambda c, i: (c, i, 0)),
            pl.BlockSpec((1, _RB, H), lambda c, i: (c, i, 0)),
            pl.BlockSpec((1, _RB, H), lambda c, i: (c, i, 0)),
            pl.BlockSpec((8, H), lambda c, i: (0, 0)),
            pl.BlockSpec((H, H), lambda c, i: (0, 0)),
        ],
        out_specs=pl.BlockSpec((1, _RB, H), lambda c, i: (c, i, 0)),
        out_shape=jax.ShapeDtypeStruct((NC, NP, H), jnp.float32),
    )(agg, hp, dinvb, vec, w)


_SEG = N // G


def _tc_final_body(agg_ref, hp_ref, dinvb_ref, vec_ref, wc_ref, avec_ref,
                   g_ref, an_ref):
    e = (agg_ref[0, 0] + hp_ref[0, 0]) * dinvb_ref[0, 0]
    e = _bn_relu(e, vec_ref[...])
    t = lax.dot_general(e, wc_ref[...], (((1,), (1,)), ((), ())),
                        preferred_element_type=jnp.float32)
    t = jnp.tanh(t + avec_ref[0:1, :])
    s = jnp.sum(t * avec_ref[1:2, :], axis=1, keepdims=True) + avec_ref[2, 0]
    m = jnp.max(s)
    w = jnp.exp(s - m)
    w = w / jnp.sum(w)
    gvec = jnp.sum(w * e, axis=0, keepdims=True)
    g_ref[0, 0] = jnp.broadcast_to(gvec, (8, H))
    nrm = jnp.sqrt(jnp.sum(e * e, axis=1, keepdims=True))
    an_ref[0, 0] = e / jnp.maximum(nrm, 1e-8)


def _tc_final(agg, hp, dinvb, vec, wc, avec):
    return pl.pallas_call(
        _tc_final_body,
        grid=(NC, G),
        in_specs=[
            pl.BlockSpec((1, 1, _SEG, H), lambda c, g: (c, g, 0, 0)),
            pl.BlockSpec((1, 1, _SEG, H), lambda c, g: (c, g, 0, 0)),
            pl.BlockSpec((1, 1, _SEG, H), lambda c, g: (c, g, 0, 0)),
            pl.BlockSpec((8, H), lambda c, g: (0, 0)),
            pl.BlockSpec((H, H), lambda c, g: (0, 0)),
            pl.BlockSpec((8, H), lambda c, g: (0, 0)),
        ],
        out_specs=[
            pl.BlockSpec((1, 1, 8, H), lambda c, g: (c, g, 0, 0)),
            pl.BlockSpec((1, 1, _SEG, H), lambda c, g: (c, g, 0, 0)),
        ],
        out_shape=[
            jax.ShapeDtypeStruct((NC, G, 8, H), jnp.float32),
            jax.ShapeDtypeStruct((NC, G, _SEG, H), jnp.float32),
        ],
    )(agg, hp, dinvb, vec, wc, avec)


def _tc_hist_body(a1_ref, a2_ref, hist_ref):
    sims = lax.dot_general(a1_ref[0, 0], a2_ref[0, 0],
                           (((1,), (1,)), ((), ())),
                           preferred_element_type=jnp.float32)
    u = (sims + 1.0) * (BINS / 2.0)
    q = jnp.clip(jnp.floor(u), 0.0, BINS - 1.0)
    lanes = lax.broadcasted_iota(jnp.int32, (1, H), 1)
    acc = jnp.zeros((1, H), jnp.float32)
    total = 0.0
    for b in range(BINS):
        cnt = jnp.sum(jnp.where(q == float(b), 1.0, 0.0))
        acc = acc + cnt * jnp.where(lanes == b, 1.0, 0.0)
        total = total + cnt
    hist_ref[0] = jnp.broadcast_to(acc / (total + 1e-8), (8, H))


def _tc_hist(an):
    return pl.pallas_call(
        _tc_hist_body,
        grid=(G,),
        in_specs=[
            pl.BlockSpec((1, 1, _SEG, H), lambda g: (0, g, 0, 0)),
            pl.BlockSpec((1, 1, _SEG, H), lambda g: (1, g, 0, 0)),
        ],
        out_specs=pl.BlockSpec((1, 8, H), lambda g: (g, 0, 0)),
        out_shape=jax.ShapeDtypeStruct((G, 8, H), jnp.float32),
    )(an, an)


def _tc_tail_body(g1_ref, g2_ref, hist_ref, T_ref, nb_ref, m1w_ref, m1b_ref,
                  m2w_ref, m2b_ref, m3w_ref, m3b_ref, h1w_ref, h1b_ref,
                  h2w_ref, h2b_ref, f1w_ref, f1b_ref, f2w_ref, f2b_ref,
                  out_ref):
    g1 = g1_ref[...]
    g2 = g2_ref[...]
    lanes16 = lax.broadcasted_iota(jnp.int32, (1, K), 1)
    tf = jnp.zeros((G, K), jnp.float32)
    for k in range(K):
        vk = lax.dot_general(g1, T_ref[k], (((1,), (0,)), ((), ())),
                             preferred_element_type=jnp.float32)
        col = jnp.sum(vk * g2, axis=1, keepdims=True)
        tf = tf + col * jnp.where(lanes16 == k, 1.0, 0.0)
    tf = tf + nb_ref[0:1, :]
    h = jnp.maximum(lax.dot_general(tf, m1w_ref[...], (((1,), (1,)), ((), ())),
                                    preferred_element_type=jnp.float32)
                    + m1b_ref[0:1, :], 0.0)
    h = jnp.maximum(lax.dot_general(h, m2w_ref[...], (((1,), (1,)), ((), ())),
                                    preferred_element_type=jnp.float32)
                    + m2b_ref[0:1, :], 0.0)
    ntn = jax.nn.sigmoid(jnp.sum(h * m3w_ref[0:1, :], axis=1, keepdims=True)
                         + m3b_ref[0, 0])
    hh = jnp.maximum(lax.dot_general(hist_ref[...], h1w_ref[...],
                                     (((1,), (1,)), ((), ())),
                                     preferred_element_type=jnp.float32)
                     + h1b_ref[0:1, :], 0.0)
    hemb = lax.dot_general(hh, h2w_ref[...], (((1,), (1,)), ((), ())),
                           preferred_element_type=jnp.float32) + h2b_ref[0:1, :]
    f1wt = f1w_ref[...]
    f = ntn * f1wt[0:1, :] \
        + lax.dot_general(hemb, lax.slice(f1wt, (1, 0), (17, 8)),
                          (((1,), (0,)), ((), ())),
                          preferred_element_type=jnp.float32)
    f = jnp.maximum(f + f1b_ref[0:1, :], 0.0)
    out = jax.nn.sigmoid(jnp.sum(f * f2w_ref[0:1, :], axis=1, keepdims=True)
                         + f2b_ref[0, 0])
    out_ref[...] = out


def _tc_tail(g1, g2, hist, p):
    args = (g1, g2, hist, p['ntn_T'], p['ntn_bias'][None, :],
            p['ntn_m1_W'], p['ntn_m1_b'][None, :],
            p['ntn_m2_W'], p['ntn_m2_b'][None, :],
            p['ntn_m3_W'], p['ntn_m3_b'][None, None],
            p['hist1_W'], p['hist1_b'][None, :],
            p['hist2_W'], p['hist2_b'][None, :],
            p['fus1_W'].T, p['fus1_b'][None, :],
            p['fus2_W'], p['fus2_b'][None, None])
    return pl.pallas_call(
        _tc_tail_body,
        out_shape=jax.ShapeDtypeStruct((G, 1), jnp.float32),
    )(*args)


def _stage_edges(ei, row_off):
    src = jnp.concatenate(
        [ei[0] + row_off,
         jnp.full((EPAD - E,), row_off, jnp.int32)]).reshape(NS, NCHUNK, CH)
    dst = jnp.concatenate(
        [ei[1], jnp.full((EPAD - E,), NP - 8, jnp.int32)]
    ).reshape(NS, NCHUNK, CH)
    pad = NCHS - NCHUNK
    src = jnp.concatenate(
        [src, jnp.full((NS, pad, CH), row_off, jnp.int32)], axis=1)
    dst = jnp.concatenate(
        [dst, jnp.full((NS, pad, CH), NP - 8, jnp.int32)], axis=1)
    return src, dst


def kernel(x1, edge_index1, batch1, x2, edge_index2, batch2, params):
    p = params
    s1, d1 = _stage_edges(edge_index1, 0)
    s2, d2 = _stage_edges(edge_index2, NP)
    src3 = jnp.concatenate([s1, s2])
    dst3 = jnp.concatenate([d1, d2])

    xs = jnp.zeros((NC, NP, D), jnp.float32)
    xs = xs.at[0, :N].set(x1).at[1, :N].set(x2)

    degtab = _make_sc_deg()(dst3).reshape(NC, NS, NP)

    def vecs(l):
        return jnp.stack([
            p['conv%d_b' % l], p['bn%d_mean' % l], p['bn%d_var' % l],
            p['bn%d_gamma' % l], p['bn%d_beta' % l],
            jnp.zeros((H,), jnp.float32), jnp.zeros((H,), jnp.float32),
            jnp.zeros((H,), jnp.float32)])

    hp0, dinvb = _tc_prep(xs, degtab, p['conv0_W'])

    def agg(hp):
        return _make_sc_agg()(src3, dst3,
                              hp.reshape(NC * NP, H)).reshape(NC, NP, H)

    agg0 = agg(hp0)
    hp1 = _tc_mid(agg0, hp0, dinvb, vecs(0), p['conv1_W'])
    agg1 = agg(hp1)
    hp2 = _tc_mid(agg1, hp1, dinvb, vecs(1), p['conv2_W'])
    agg2 = agg(hp2)

    avec = jnp.zeros((8, H), jnp.float32)
    avec = avec.at[0].set(p['attn_cw_b']).at[1].set(p['attn_a_W'][0])
    avec = avec.at[2, 0].set(p['attn_a_b'][0])

    def seg4(a):
        return a[:, :N].reshape(NC, G, _SEG, H)
    gout, an = _tc_final(seg4(agg2), seg4(hp2), seg4(dinvb), vecs(2),
                         p['attn_cw_W'], avec)

    hist = _tc_hist(an)[:, 0, :BINS]
    g1 = gout[0, :, 0, :]
    g2 = gout[1, :, 0, :]
    return _tc_tail(g1, g2, hist, p)


# speedup vs baseline: 1.2957x; 1.2129x over previous
import functools

import jax
import jax.numpy as jnp
from jax import lax
from jax.experimental import pallas as pl
from jax.experimental.pallas import tpu as pltpu
from jax.experimental.pallas import tpu_sc as plsc

N = 10000
E = 320000
D = 128
H = 128
G = 20
K = 16
BINS = 16

NP = 10240
NC, NS = 2, 16
CH = 160
EPT = 20000
EPAD = NS * EPT
NCHUNK = EPT // CH
NCH2 = 126
NCHS = 128
RPT = NP // NS

_mesh = functools.partial(
    plsc.VectorSubcoreMesh, core_axis_name="c", subcore_axis_name="s",
    num_cores=NC, num_subcores=NS)


@functools.cache
def _make_sc_deg():
    @functools.partial(
        pl.kernel,
        out_type=jax.ShapeDtypeStruct((NC * NS * NP,), jnp.float32),
        mesh=_mesh(),
        compiler_params=pltpu.CompilerParams(needs_layout_passes=False),
        scratch_types=[
            pltpu.VMEM((NCHS, CH), jnp.int32),
            pltpu.VMEM((NP,), jnp.float32),
        ],
    )
    def _sc_deg(dst3_hbm, deg_hbm, dstb_v, degtab_v):
        c = lax.axis_index("c")
        s = lax.axis_index("s")
        w = c * NS + s

        zero16 = jnp.zeros((16,), jnp.float32)

        def _zero(i, _):
            degtab_v[pl.ds(i * 16, 16)] = zero16
            return 0
        lax.fori_loop(0, NP // 16, _zero, 0)

        pltpu.sync_copy(dst3_hbm.at[w], dstb_v)
        one16 = jnp.full((16,), 1.0, jnp.float32)

        def _chunk(i, _):
            def _inner(j, _):
                d16 = dstb_v[i, pl.ds(j * 16, 16)]
                plsc.addupdate_scatter(degtab_v, [d16], one16)
                return 0
            lax.fori_loop(0, CH // 16, _inner, 0)
            return 0
        lax.fori_loop(0, NCHUNK, _chunk, 0)

        pltpu.sync_copy(degtab_v, deg_hbm.at[pl.ds((c * NS + s) * NP, NP)])

    return _sc_deg


@functools.cache
def _make_sc_agg():
    @functools.partial(
        pl.kernel,
        out_type=jax.ShapeDtypeStruct((NC * NP, H), jnp.float32),
        mesh=_mesh(),
        compiler_params=pltpu.CompilerParams(needs_layout_passes=False),
        scratch_types=[
            pltpu.VMEM((1, CH), jnp.int32),
            pltpu.VMEM((1, CH), jnp.int32),
            pltpu.VMEM((1, CH), jnp.int32),
            pltpu.VMEM((1, CH), jnp.int32),
            pltpu.VMEM((CH, H), jnp.float32),
            pltpu.VMEM((CH, H), jnp.float32),
            pltpu.VMEM_SHARED((NP, H), jnp.float32),
            pltpu.SemaphoreType.DMA,
            pltpu.SemaphoreType.DMA,
            pltpu.SemaphoreType.DMA,
            pltpu.SemaphoreType.DMA,
        ],
    )
    def _sc_agg(src3_hbm, dst3_hbm, hp_hbm, out_hbm, srcA, dstA, srcB, dstB,
                rowsA, rowsB, acc_sh, isemA, isemB, gsemA, gsemB):
        c = lax.axis_index("c")
        s = lax.axis_index("s")
        w = c * NS + s

        zero16 = jnp.zeros((16,), jnp.float32)

        def _zrow(i, _):
            for l in range(H // 16):
                rowsA[i, pl.ds(l * 16, 16)] = zero16
            return 0
        lax.fori_loop(0, CH, _zrow, 0)
        for r in range(RPT // CH):
            pltpu.sync_copy(rowsA, acc_sh.at[pl.ds(s * RPT + r * CH, CH)])
        plsc.subcore_barrier()

        def fetch_idx(i, sb, db, sem):
            pltpu.async_copy(src3_hbm.at[w, pl.ds(i, 1)], sb, sem)
            pltpu.async_copy(dst3_hbm.at[w, pl.ds(i, 1)], db, sem)

        def wait_idx(i, sb, db, sem):
            pltpu.make_async_copy(src3_hbm.at[w, pl.ds(i, 1)], sb, sem).wait()
            pltpu.make_async_copy(dst3_hbm.at[w, pl.ds(i, 1)], db, sem).wait()

        wait = pltpu.make_async_copy
        fetch_idx(0, srcA, dstA, isemA)
        fetch_idx(1, srcB, dstB, isemB)
        wait_idx(0, srcA, dstA, isemA)
        pltpu.async_copy(hp_hbm.at[srcA.at[0]], rowsA, gsemA)

        def _chunk2(i2, _):
            i = 2 * i2
            wait_idx(i + 1, srcB, dstB, isemB)
            pltpu.async_copy(hp_hbm.at[srcB.at[0]], rowsB, gsemB)
            wait(hp_hbm.at[srcA.at[0]], rowsA, gsemA).wait()
            pltpu.sync_copy(rowsA, acc_sh.at[dstA.at[0]], add=True)
            fetch_idx(i + 2, srcA, dstA, isemA)
            wait(hp_hbm.at[srcB.at[0]], rowsB, gsemB).wait()
            pltpu.sync_copy(rowsB, acc_sh.at[dstB.at[0]], add=True)
            fetch_idx(i + 3, srcB, dstB, isemB)
            wait_idx(i + 2, srcA, dstA, isemA)
            pltpu.async_copy(hp_hbm.at[srcA.at[0]], rowsA, gsemA)
            return 0
        lax.fori_loop(0, NCH2 // 2, _chunk2, 0)

        wait(hp_hbm.at[srcA.at[0]], rowsA, gsemA).wait()
        wait_idx(NCH2 + 1, srcB, dstB, isemB)

        plsc.subcore_barrier()
        pltpu.sync_copy(acc_sh.at[pl.ds(s * RPT, RPT)],
                        out_hbm.at[pl.ds(c * NP + s * RPT, RPT)])

    return _sc_agg


_RB = 2048
_NRB = NP // _RB


def _tc_prep_body(x_ref, tab_ref, w_ref, hp_ref, dinvb_ref):
    tab = tab_ref[0]
    ones = jnp.ones((NS, 1), jnp.float32)
    deg = 1.0 + lax.dot_general(tab, ones, (((0,), (0,)), ((), ())),
                                preferred_element_type=jnp.float32)
    dinv = lax.rsqrt(deg)
    dinvb = jnp.broadcast_to(dinv, (_RB, H))
    h = lax.dot_general(x_ref[0], w_ref[...], (((1,), (1,)), ((), ())),
                        preferred_element_type=jnp.float32)
    hp_ref[0] = h * dinvb
    dinvb_ref[0] = dinvb


def _tc_prep(xs, degtab, w0):
    return pl.pallas_call(
        _tc_prep_body,
        grid=(NC, _NRB),
        in_specs=[
            pl.BlockSpec((1, _RB, D), lambda c, i: (c, i, 0)),
            pl.BlockSpec((1, NS, _RB), lambda c, i: (c, 0, i)),
            pl.BlockSpec((H, D), lambda c, i: (0, 0)),
        ],
        out_specs=[
            pl.BlockSpec((1, _RB, H), lambda c, i: (c, i, 0)),
            pl.BlockSpec((1, _RB, H), lambda c, i: (c, i, 0)),
        ],
        out_shape=[
            jax.ShapeDtypeStruct((NC, NP, H), jnp.float32),
            jax.ShapeDtypeStruct((NC, NP, H), jnp.float32),
        ],
    )(xs, degtab, w0)


def _bn_relu(h, vec):
    h = h + vec[0:1, :]
    h = (h - vec[1:2, :]) * lax.rsqrt(vec[2:3, :] + 1e-5) * vec[3:4, :] \
        + vec[4:5, :]
    return jnp.maximum(h, 0.0)


def _tc_mid_body(agg_ref, hp_ref, dinvb_ref, vec_ref, w_ref, out_ref):
    dinvb = dinvb_ref[0]
    h = (agg_ref[0] + hp_ref[0]) * dinvb
    h = _bn_relu(h, vec_ref[...])
    hn = lax.dot_general(h, w_ref[...], (((1,), (1,)), ((), ())),
                         preferred_element_type=jnp.float32)
    out_ref[0] = hn * dinvb


def _tc_mid(agg, hp, dinvb, vec, w):
    return pl.pallas_call(
        _tc_mid_body,
        grid=(NC, _NRB),
        in_specs=[
            pl.BlockSpec((1, _RB, H), lambda c, i: (c, i, 0)),
            pl.BlockSpec((1, _RB, H), lambda c, i: (c, i, 0)),
            pl.BlockSpec((1, _RB, H), lambda c, i: (c, i, 0)),
            pl.BlockSpec((8, H), lambda c, i: (0, 0)),
            pl.BlockSpec((H, H), lambda c, i: (0, 0)),
        ],
        out_specs=pl.BlockSpec((1, _RB, H), lambda c, i: (c, i, 0)),
        out_shape=jax.ShapeDtypeStruct((NC, NP, H), jnp.float32),
    )(agg, hp, dinvb, vec, w)


_SEG = N // G


def _tc_final_body(agg_ref, hp_ref, dinvb_ref, vec_ref, wc_ref, avec_ref,
                   g_ref, an_ref):
    e = (agg_ref[0, 0] + hp_ref[0, 0]) * dinvb_ref[0, 0]
    e = _bn_relu(e, vec_ref[...])
    t = lax.dot_general(e, wc_ref[...], (((1,), (1,)), ((), ())),
                        preferred_element_type=jnp.float32)
    t = jnp.tanh(t + avec_ref[0:1, :])
    s = jnp.sum(t * avec_ref[1:2, :], axis=1, keepdims=True) + avec_ref[2, 0]
    m = jnp.max(s)
    w = jnp.exp(s - m)
    w = w / jnp.sum(w)
    gvec = jnp.sum(w * e, axis=0, keepdims=True)
    g_ref[0, 0] = jnp.broadcast_to(gvec, (8, H))
    nrm = jnp.sqrt(jnp.sum(e * e, axis=1, keepdims=True))
    an_ref[0, 0] = e / jnp.maximum(nrm, 1e-8)


def _tc_final(agg, hp, dinvb, vec, wc, avec):
    return pl.pallas_call(
        _tc_final_body,
        grid=(NC, G),
        in_specs=[
            pl.BlockSpec((1, 1, _SEG, H), lambda c, g: (c, g, 0, 0)),
            pl.BlockSpec((1, 1, _SEG, H), lambda c, g: (c, g, 0, 0)),
            pl.BlockSpec((1, 1, _SEG, H), lambda c, g: (c, g, 0, 0)),
            pl.BlockSpec((8, H), lambda c, g: (0, 0)),
            pl.BlockSpec((H, H), lambda c, g: (0, 0)),
            pl.BlockSpec((8, H), lambda c, g: (0, 0)),
        ],
        out_specs=[
            pl.BlockSpec((1, 1, 8, H), lambda c, g: (c, g, 0, 0)),
            pl.BlockSpec((1, 1, _SEG, H), lambda c, g: (c, g, 0, 0)),
        ],
        out_shape=[
            jax.ShapeDtypeStruct((NC, G, 8, H), jnp.float32),
            jax.ShapeDtypeStruct((NC, G, _SEG, H), jnp.float32),
        ],
    )(agg, hp, dinvb, vec, wc, avec)


def _tc_hist_body(a1_ref, a2_ref, hist_ref):
    sims = lax.dot_general(a1_ref[0, 0], a2_ref[0, 0],
                           (((1,), (1,)), ((), ())),
                           preferred_element_type=jnp.float32)
    u = (sims + 1.0) * (BINS / 2.0)
    q = jnp.clip(jnp.floor(u), 0.0, BINS - 1.0)
    lanes = lax.broadcasted_iota(jnp.int32, (1, H), 1)
    acc = jnp.zeros((1, H), jnp.float32)
    total = 0.0
    for b in range(BINS):
        cnt = jnp.sum(jnp.where(q == float(b), 1.0, 0.0))
        acc = acc + cnt * jnp.where(lanes == b, 1.0, 0.0)
        total = total + cnt
    hist_ref[0] = jnp.broadcast_to(acc / (total + 1e-8), (8, H))


def _tc_hist(an):
    return pl.pallas_call(
        _tc_hist_body,
        grid=(G,),
        in_specs=[
            pl.BlockSpec((1, 1, _SEG, H), lambda g: (0, g, 0, 0)),
            pl.BlockSpec((1, 1, _SEG, H), lambda g: (1, g, 0, 0)),
        ],
        out_specs=pl.BlockSpec((1, 8, H), lambda g: (g, 0, 0)),
        out_shape=jax.ShapeDtypeStruct((G, 8, H), jnp.float32),
    )(an, an)


def _tc_tail_body(g1_ref, g2_ref, hist_ref, T_ref, nb_ref, m1w_ref, m1b_ref,
                  m2w_ref, m2b_ref, m3w_ref, m3b_ref, h1w_ref, h1b_ref,
                  h2w_ref, h2b_ref, f1w_ref, f1b_ref, f2w_ref, f2b_ref,
                  out_ref):
    g1 = g1_ref[...]
    g2 = g2_ref[...]
    lanes16 = lax.broadcasted_iota(jnp.int32, (1, K), 1)
    tf = jnp.zeros((G, K), jnp.float32)
    for k in range(K):
        vk = lax.dot_general(g1, T_ref[k], (((1,), (0,)), ((), ())),
                             preferred_element_type=jnp.float32)
        col = jnp.sum(vk * g2, axis=1, keepdims=True)
        tf = tf + col * jnp.where(lanes16 == k, 1.0, 0.0)
    tf = tf + nb_ref[0:1, :]
    h = jnp.maximum(lax.dot_general(tf, m1w_ref[...], (((1,), (1,)), ((), ())),
                                    preferred_element_type=jnp.float32)
                    + m1b_ref[0:1, :], 0.0)
    h = jnp.maximum(lax.dot_general(h, m2w_ref[...], (((1,), (1,)), ((), ())),
                                    preferred_element_type=jnp.float32)
                    + m2b_ref[0:1, :], 0.0)
    ntn = jax.nn.sigmoid(jnp.sum(h * m3w_ref[0:1, :], axis=1, keepdims=True)
                         + m3b_ref[0, 0])
    hh = jnp.maximum(lax.dot_general(hist_ref[...], h1w_ref[...],
                                     (((1,), (1,)), ((), ())),
                                     preferred_element_type=jnp.float32)
                     + h1b_ref[0:1, :], 0.0)
    hemb = lax.dot_general(hh, h2w_ref[...], (((1,), (1,)), ((), ())),
                           preferred_element_type=jnp.float32) + h2b_ref[0:1, :]
    f1wt = f1w_ref[...]
    f = ntn * f1wt[0:1, :] \
        + lax.dot_general(hemb, lax.slice(f1wt, (1, 0), (17, 8)),
                          (((1,), (0,)), ((), ())),
                          preferred_element_type=jnp.float32)
    f = jnp.maximum(f + f1b_ref[0:1, :], 0.0)
    out = jax.nn.sigmoid(jnp.sum(f * f2w_ref[0:1, :], axis=1, keepdims=True)
                         + f2b_ref[0, 0])
    out_ref[...] = out


def _tc_tail(g1, g2, hist, p):
    args = (g1, g2, hist, p['ntn_T'], p['ntn_bias'][None, :],
            p['ntn_m1_W'], p['ntn_m1_b'][None, :],
            p['ntn_m2_W'], p['ntn_m2_b'][None, :],
            p['ntn_m3_W'], p['ntn_m3_b'][None, None],
            p['hist1_W'], p['hist1_b'][None, :],
            p['hist2_W'], p['hist2_b'][None, :],
            p['fus1_W'].T, p['fus1_b'][None, :],
            p['fus2_W'], p['fus2_b'][None, None])
    return pl.pallas_call(
        _tc_tail_body,
        out_shape=jax.ShapeDtypeStruct((G, 1), jnp.float32),
    )(*args)


def _stage_edges(ei, row_off):
    src = jnp.concatenate(
        [ei[0] + row_off,
         jnp.full((EPAD - E,), row_off, jnp.int32)]).reshape(NS, NCHUNK, CH)
    dst = jnp.concatenate(
        [ei[1], jnp.full((EPAD - E,), NP - 8, jnp.int32)]
    ).reshape(NS, NCHUNK, CH)
    pad = NCHS - NCHUNK
    src = jnp.concatenate(
        [src, jnp.full((NS, pad, CH), row_off, jnp.int32)], axis=1)
    dst = jnp.concatenate(
        [dst, jnp.full((NS, pad, CH), NP - 8, jnp.int32)], axis=1)
    return src, dst


def kernel(x1, edge_index1, batch1, x2, edge_index2, batch2, params):
    p = params
    s1, d1 = _stage_edges(edge_index1, 0)
    s2, d2 = _stage_edges(edge_index2, NP)
    src3 = jnp.concatenate([s1, s2])
    dst3 = jnp.concatenate([d1, d2])

    xs = jnp.zeros((NC, NP, D), jnp.float32)
    xs = xs.at[0, :N].set(x1).at[1, :N].set(x2)

    degtab = _make_sc_deg()(dst3).reshape(NC, NS, NP)

    def vecs(l):
        return jnp.stack([
            p['conv%d_b' % l], p['bn%d_mean' % l], p['bn%d_var' % l],
            p['bn%d_gamma' % l], p['bn%d_beta' % l],
            jnp.zeros((H,), jnp.float32), jnp.zeros((H,), jnp.float32),
            jnp.zeros((H,), jnp.float32)])

    hp0, dinvb = _tc_prep(xs, degtab, p['conv0_W'])

    def agg(hp):
        return _make_sc_agg()(src3, dst3,
                              hp.reshape(NC * NP, H)).reshape(NC, NP, H)

    agg0 = agg(hp0)
    hp1 = _tc_mid(agg0, hp0, dinvb, vecs(0), p['conv1_W'])
    agg1 = agg(hp1)
    hp2 = _tc_mid(agg1, hp1, dinvb, vecs(1), p['conv2_W'])
    agg2 = agg(hp2)

    avec = jnp.zeros((8, H), jnp.float32)
    avec = avec.at[0].set(p['attn_cw_b']).at[1].set(p['attn_a_W'][0])
    avec = avec.at[2, 0].set(p['attn_a_b'][0])

    def seg4(a):
        return a[:, :N].reshape(NC, G, _SEG, H)
    gout, an = _tc_final(seg4(agg2), seg4(hp2), seg4(dinvb), vecs(2),
                         p['attn_cw_W'], avec)

    hist = _tc_hist(an)[:, 0, :BINS]
    g1 = gout[0, :, 0, :]
    g2 = gout[1, :, 0, :]
    return _tc_tail(g1, g2, hist, p)


# speedup vs baseline: 1.2972x; 1.0011x over previous
import functools

import jax
import jax.numpy as jnp
from jax import lax
from jax.experimental import pallas as pl
from jax.experimental.pallas import tpu as pltpu
from jax.experimental.pallas import tpu_sc as plsc

N = 10000
E = 320000
D = 128
H = 128
G = 20
K = 16
BINS = 16

NP = 10240
NC, NS = 2, 16
CH = 160
EPT = 20000
EPAD = NS * EPT
NCHUNK = EPT // CH
NCH2 = 126
NCHS = 128
RPT = NP // NS

_mesh = functools.partial(
    plsc.VectorSubcoreMesh, core_axis_name="c", subcore_axis_name="s",
    num_cores=NC, num_subcores=NS)


@functools.cache
def _make_sc_deg():
    @functools.partial(
        pl.kernel,
        out_type=jax.ShapeDtypeStruct((NC * NS * NP,), jnp.float32),
        mesh=_mesh(),
        compiler_params=pltpu.CompilerParams(needs_layout_passes=False),
        scratch_types=[
            pltpu.VMEM((NCHS, CH), jnp.int32),
            pltpu.VMEM((NP,), jnp.float32),
        ],
    )
    def _sc_deg(dst3_hbm, deg_hbm, dstb_v, degtab_v):
        c = lax.axis_index("c")
        s = lax.axis_index("s")
        w = c * NS + s

        zero16 = jnp.zeros((16,), jnp.float32)

        def _zero(i, _):
            degtab_v[pl.ds(i * 16, 16)] = zero16
            return 0
        lax.fori_loop(0, NP // 16, _zero, 0)

        pltpu.sync_copy(dst3_hbm.at[w], dstb_v)
        one16 = jnp.full((16,), 1.0, jnp.float32)

        def _chunk(i, _):
            def _inner(j, _):
                d16 = dstb_v[i, pl.ds(j * 16, 16)]
                plsc.addupdate_scatter(degtab_v, [d16], one16)
                return 0
            lax.fori_loop(0, CH // 16, _inner, 0)
            return 0
        lax.fori_loop(0, NCHUNK, _chunk, 0)

        pltpu.sync_copy(degtab_v, deg_hbm.at[pl.ds((c * NS + s) * NP, NP)])

    return _sc_deg


@functools.cache
def _make_sc_agg():
    @functools.partial(
        pl.kernel,
        out_type=jax.ShapeDtypeStruct((NC * NP, H), jnp.float32),
        mesh=_mesh(),
        compiler_params=pltpu.CompilerParams(needs_layout_passes=False),
        scratch_types=[
            pltpu.VMEM((1, CH), jnp.int32),
            pltpu.VMEM((1, CH), jnp.int32),
            pltpu.VMEM((1, CH), jnp.int32),
            pltpu.VMEM((1, CH), jnp.int32),
            pltpu.VMEM((CH, H), jnp.float32),
            pltpu.VMEM((CH, H), jnp.float32),
            pltpu.VMEM_SHARED((NP, H), jnp.float32),
            pltpu.SemaphoreType.DMA,
            pltpu.SemaphoreType.DMA,
            pltpu.SemaphoreType.DMA,
            pltpu.SemaphoreType.DMA,
        ],
    )
    def _sc_agg(src3_hbm, dst3_hbm, hp_hbm, out_hbm, srcA, dstA, srcB, dstB,
                rowsA, rowsB, acc_sh, isemA, isemB, gsemA, gsemB):
        c = lax.axis_index("c")
        s = lax.axis_index("s")
        w = c * NS + s

        zero16 = jnp.zeros((16,), jnp.float32)

        def _zrow(i, _):
            for l in range(H // 16):
                rowsA[i, pl.ds(l * 16, 16)] = zero16
            return 0
        lax.fori_loop(0, CH, _zrow, 0)
        for r in range(RPT // CH):
            pltpu.sync_copy(rowsA, acc_sh.at[pl.ds(s * RPT + r * CH, CH)])
        plsc.subcore_barrier()

        def fetch_idx(i, sb, db, sem):
            pltpu.async_copy(src3_hbm.at[w, pl.ds(i, 1)], sb, sem)
            pltpu.async_copy(dst3_hbm.at[w, pl.ds(i, 1)], db, sem)

        def wait_idx(i, sb, db, sem):
            pltpu.make_async_copy(src3_hbm.at[w, pl.ds(i, 1)], sb, sem).wait()
            pltpu.make_async_copy(dst3_hbm.at[w, pl.ds(i, 1)], db, sem).wait()

        def wait_rows(buf, sem):
            pltpu.make_async_copy(hp_hbm.at[pl.ds(0, CH)], buf, sem).wait()

        fetch_idx(0, srcA, dstA, isemA)
        fetch_idx(1, srcB, dstB, isemB)
        wait_idx(0, srcA, dstA, isemA)
        pltpu.async_copy(hp_hbm.at[srcA.at[0]], rowsA, gsemA)

        def _chunk2(i2, _):
            i = 2 * i2
            wait_idx(i + 1, srcB, dstB, isemB)
            pltpu.async_copy(hp_hbm.at[srcB.at[0]], rowsB, gsemB)
            wait_rows(rowsA, gsemA)
            pltpu.sync_copy(rowsA, acc_sh.at[dstA.at[0]], add=True)
            fetch_idx(i + 2, srcA, dstA, isemA)
            wait_rows(rowsB, gsemB)
            pltpu.sync_copy(rowsB, acc_sh.at[dstB.at[0]], add=True)
            fetch_idx(i + 3, srcB, dstB, isemB)
            wait_idx(i + 2, srcA, dstA, isemA)
            pltpu.async_copy(hp_hbm.at[srcA.at[0]], rowsA, gsemA)
            return 0
        lax.fori_loop(0, NCH2 // 2, _chunk2, 0)

        wait_rows(rowsA, gsemA)
        wait_idx(NCH2 + 1, srcB, dstB, isemB)

        plsc.subcore_barrier()
        pltpu.sync_copy(acc_sh.at[pl.ds(s * RPT, RPT)],
                        out_hbm.at[pl.ds(c * NP + s * RPT, RPT)])

    return _sc_agg


_RB = 2048
_NRB = NP // _RB


def _tc_prep_body(x_ref, tab_ref, w_ref, hp_ref, dinvb_ref):
    tab = tab_ref[0]
    ones = jnp.ones((NS, 1), jnp.float32)
    deg = 1.0 + lax.dot_general(tab, ones, (((0,), (0,)), ((), ())),
                                preferred_element_type=jnp.float32)
    dinv = lax.rsqrt(deg)
    dinvb = jnp.broadcast_to(dinv, (_RB, H))
    h = lax.dot_general(x_ref[0], w_ref[...], (((1,), (1,)), ((), ())),
                        preferred_element_type=jnp.float32)
    hp_ref[0] = h * dinvb
    dinvb_ref[0] = dinvb


def _tc_prep(xs, degtab, w0):
    return pl.pallas_call(
        _tc_prep_body,
        grid=(NC, _NRB),
        in_specs=[
            pl.BlockSpec((1, _RB, D), lambda c, i: (c, i, 0)),
            pl.BlockSpec((1, NS, _RB), lambda c, i: (c, 0, i)),
            pl.BlockSpec((H, D), lambda c, i: (0, 0)),
        ],
        out_specs=[
            pl.BlockSpec((1, _RB, H), lambda c, i: (c, i, 0)),
            pl.BlockSpec((1, _RB, H), lambda c, i: (c, i, 0)),
        ],
        out_shape=[
            jax.ShapeDtypeStruct((NC, NP, H), jnp.float32),
            jax.ShapeDtypeStruct((NC, NP, H), jnp.float32),
        ],
    )(xs, degtab, w0)


def _bn_relu(h, vec):
    h = h + vec[0:1, :]
    h = (h - vec[1:2, :]) * lax.rsqrt(vec[2:3, :] + 1e-5) * vec[3:4, :] \
        + vec[4:5, :]
    return jnp.maximum(h, 0.0)


def _tc_mid_body(agg_ref, hp_ref, dinvb_ref, vec_ref, w_ref, out_ref):
    dinvb = dinvb_ref[0]
    h = (agg_ref[0] + hp_ref[0]) * dinvb
    h = _bn_relu(h, vec_ref[...])
    hn = lax.dot_general(h, w_ref[...], (((1,), (1,)), ((), ())),
                         preferred_element_type=jnp.float32)
    out_ref[0] = hn * dinvb


def _tc_mid(agg, hp, dinvb, vec, w):
    return pl.pallas_call(
        _tc_mid_body,
        grid=(NC, _NRB),
        in_specs=[
            pl.BlockSpec((1, _RB, H), lambda c, i: (c, i, 0)),
            pl.BlockSpec((1, _RB, H), lambda c, i: (c, i, 0)),
            pl.BlockSpec((1, _RB, H), lambda c, i: (c, i, 0)),
            pl.BlockSpec((8, H), lambda c, i: (0, 0)),
            pl.BlockSpec((H, H), lambda c, i: (0, 0)),
        ],
        out_specs=pl.BlockSpec((1, _RB, H), lambda c, i: (c, i, 0)),
        out_shape=jax.ShapeDtypeStruct((NC, NP, H), jnp.float32),
    )(agg, hp, dinvb, vec, w)


_SEG = N // G


def _tc_final_body(agg_ref, hp_ref, dinvb_ref, vec_ref, wc_ref, avec_ref,
                   g_ref, an_ref):
    e = (agg_ref[0, 0] + hp_ref[0, 0]) * dinvb_ref[0, 0]
    e = _bn_relu(e, vec_ref[...])
    t = lax.dot_general(e, wc_ref[...], (((1,), (1,)), ((), ())),
                        preferred_element_type=jnp.float32)
    t = jnp.tanh(t + avec_ref[0:1, :])
    s = jnp.sum(t * avec_ref[1:2, :], axis=1, keepdims=True) + avec_ref[2, 0]
    m = jnp.max(s)
    w = jnp.exp(s - m)
    w = w / jnp.sum(w)
    gvec = jnp.sum(w * e, axis=0, keepdims=True)
    g_ref[0, 0] = jnp.broadcast_to(gvec, (8, H))
    nrm = jnp.sqrt(jnp.sum(e * e, axis=1, keepdims=True))
    an_ref[0, 0] = e / jnp.maximum(nrm, 1e-8)


def _tc_final(agg, hp, dinvb, vec, wc, avec):
    return pl.pallas_call(
        _tc_final_body,
        grid=(NC, G),
        in_specs=[
            pl.BlockSpec((1, 1, _SEG, H), lambda c, g: (c, g, 0, 0)),
            pl.BlockSpec((1, 1, _SEG, H), lambda c, g: (c, g, 0, 0)),
            pl.BlockSpec((1, 1, _SEG, H), lambda c, g: (c, g, 0, 0)),
            pl.BlockSpec((8, H), lambda c, g: (0, 0)),
            pl.BlockSpec((H, H), lambda c, g: (0, 0)),
            pl.BlockSpec((8, H), lambda c, g: (0, 0)),
        ],
        out_specs=[
            pl.BlockSpec((1, 1, 8, H), lambda c, g: (c, g, 0, 0)),
            pl.BlockSpec((1, 1, _SEG, H), lambda c, g: (c, g, 0, 0)),
        ],
        out_shape=[
            jax.ShapeDtypeStruct((NC, G, 8, H), jnp.float32),
            jax.ShapeDtypeStruct((NC, G, _SEG, H), jnp.float32),
        ],
    )(agg, hp, dinvb, vec, wc, avec)


def _tc_hist_body(a1_ref, a2_ref, hist_ref):
    sims = lax.dot_general(a1_ref[0, 0], a2_ref[0, 0],
                           (((1,), (1,)), ((), ())),
                           preferred_element_type=jnp.float32)
    u = (sims + 1.0) * (BINS / 2.0)
    q = jnp.clip(jnp.floor(u), 0.0, BINS - 1.0)
    lanes = lax.broadcasted_iota(jnp.int32, (1, H), 1)
    acc = jnp.zeros((1, H), jnp.float32)
    total = 0.0
    for b in range(BINS):
        cnt = jnp.sum(jnp.where(q == float(b), 1.0, 0.0))
        acc = acc + cnt * jnp.where(lanes == b, 1.0, 0.0)
        total = total + cnt
    hist_ref[0] = jnp.broadcast_to(acc / (total + 1e-8), (8, H))


def _tc_hist(an):
    return pl.pallas_call(
        _tc_hist_body,
        grid=(G,),
        in_specs=[
            pl.BlockSpec((1, 1, _SEG, H), lambda g: (0, g, 0, 0)),
            pl.BlockSpec((1, 1, _SEG, H), lambda g: (1, g, 0, 0)),
        ],
        out_specs=pl.BlockSpec((1, 8, H), lambda g: (g, 0, 0)),
        out_shape=jax.ShapeDtypeStruct((G, 8, H), jnp.float32),
    )(an, an)


def _tc_tail_body(g1_ref, g2_ref, hist_ref, T_ref, nb_ref, m1w_ref, m1b_ref,
                  m2w_ref, m2b_ref, m3w_ref, m3b_ref, h1w_ref, h1b_ref,
                  h2w_ref, h2b_ref, f1w_ref, f1b_ref, f2w_ref, f2b_ref,
                  out_ref):
    g1 = g1_ref[...]
    g2 = g2_ref[...]
    lanes16 = lax.broadcasted_iota(jnp.int32, (1, K), 1)
    tf = jnp.zeros((G, K), jnp.float32)
    for k in range(K):
        vk = lax.dot_general(g1, T_ref[k], (((1,), (0,)), ((), ())),
                             preferred_element_type=jnp.float32)
        col = jnp.sum(vk * g2, axis=1, keepdims=True)
        tf = tf + col * jnp.where(lanes16 == k, 1.0, 0.0)
    tf = tf + nb_ref[0:1, :]
    h = jnp.maximum(lax.dot_general(tf, m1w_ref[...], (((1,), (1,)), ((), ())),
                                    preferred_element_type=jnp.float32)
                    + m1b_ref[0:1, :], 0.0)
    h = jnp.maximum(lax.dot_general(h, m2w_ref[...], (((1,), (1,)), ((), ())),
                                    preferred_element_type=jnp.float32)
                    + m2b_ref[0:1, :], 0.0)
    ntn = jax.nn.sigmoid(jnp.sum(h * m3w_ref[0:1, :], axis=1, keepdims=True)
                         + m3b_ref[0, 0])
    hh = jnp.maximum(lax.dot_general(hist_ref[...], h1w_ref[...],
                                     (((1,), (1,)), ((), ())),
                                     preferred_element_type=jnp.float32)
                     + h1b_ref[0:1, :], 0.0)
    hemb = lax.dot_general(hh, h2w_ref[...], (((1,), (1,)), ((), ())),
                           preferred_element_type=jnp.float32) + h2b_ref[0:1, :]
    f1wt = f1w_ref[...]
    f = ntn * f1wt[0:1, :] \
        + lax.dot_general(hemb, lax.slice(f1wt, (1, 0), (17, 8)),
                          (((1,), (0,)), ((), ())),
                          preferred_element_type=jnp.float32)
    f = jnp.maximum(f + f1b_ref[0:1, :], 0.0)
    out = jax.nn.sigmoid(jnp.sum(f * f2w_ref[0:1, :], axis=1, keepdims=True)
                         + f2b_ref[0, 0])
    out_ref[...] = out


def _tc_tail(g1, g2, hist, p):
    args = (g1, g2, hist, p['ntn_T'], p['ntn_bias'][None, :],
            p['ntn_m1_W'], p['ntn_m1_b'][None, :],
            p['ntn_m2_W'], p['ntn_m2_b'][None, :],
            p['ntn_m3_W'], p['ntn_m3_b'][None, None],
            p['hist1_W'], p['hist1_b'][None, :],
            p['hist2_W'], p['hist2_b'][None, :],
            p['fus1_W'].T, p['fus1_b'][None, :],
            p['fus2_W'], p['fus2_b'][None, None])
    return pl.pallas_call(
        _tc_tail_body,
        out_shape=jax.ShapeDtypeStruct((G, 1), jnp.float32),
    )(*args)


def _stage_edges(ei, row_off):
    src = jnp.concatenate(
        [ei[0] + row_off,
         jnp.full((EPAD - E,), row_off, jnp.int32)]).reshape(NS, NCHUNK, CH)
    dst = jnp.concatenate(
        [ei[1], jnp.full((EPAD - E,), NP - 8, jnp.int32)]
    ).reshape(NS, NCHUNK, CH)
    pad = NCHS - NCHUNK
    src = jnp.concatenate(
        [src, jnp.full((NS, pad, CH), row_off, jnp.int32)], axis=1)
    dst = jnp.concatenate(
        [dst, jnp.full((NS, pad, CH), NP - 8, jnp.int32)], axis=1)
    return src, dst


def kernel(x1, edge_index1, batch1, x2, edge_index2, batch2, params):
    p = params
    s1, d1 = _stage_edges(edge_index1, 0)
    s2, d2 = _stage_edges(edge_index2, NP)
    src3 = jnp.concatenate([s1, s2])
    dst3 = jnp.concatenate([d1, d2])

    xs = jnp.zeros((NC, NP, D), jnp.float32)
    xs = xs.at[0, :N].set(x1).at[1, :N].set(x2)

    degtab = _make_sc_deg()(dst3).reshape(NC, NS, NP)

    def vecs(l):
        return jnp.stack([
            p['conv%d_b' % l], p['bn%d_mean' % l], p['bn%d_var' % l],
            p['bn%d_gamma' % l], p['bn%d_beta' % l],
            jnp.zeros((H,), jnp.float32), jnp.zeros((H,), jnp.float32),
            jnp.zeros((H,), jnp.float32)])

    hp0, dinvb = _tc_prep(xs, degtab, p['conv0_W'])

    def agg(hp):
        return _make_sc_agg()(src3, dst3,
                              hp.reshape(NC * NP, H)).reshape(NC, NP, H)

    agg0 = agg(hp0)
    hp1 = _tc_mid(agg0, hp0, dinvb, vecs(0), p['conv1_W'])
    agg1 = agg(hp1)
    hp2 = _tc_mid(agg1, hp1, dinvb, vecs(1), p['conv2_W'])
    agg2 = agg(hp2)

    avec = jnp.zeros((8, H), jnp.float32)
    avec = avec.at[0].set(p['attn_cw_b']).at[1].set(p['attn_a_W'][0])
    avec = avec.at[2, 0].set(p['attn_a_b'][0])

    def seg4(a):
        return a[:, :N].reshape(NC, G, _SEG, H)
    gout, an = _tc_final(seg4(agg2), seg4(hp2), seg4(dinvb), vecs(2),
                         p['attn_cw_W'], avec)

    hist = _tc_hist(an)[:, 0, :BINS]
    g1 = gout[0, :, 0, :]
    g2 = gout[1, :, 0, :]
    return _tc_tail(g1, g2, hist, p)
